# Initial kernel scaffold; baseline (speedup 1.0000x reference)
import jax
import jax.numpy as jnp
from jax.experimental import pallas as pl


def kernel(points, W, b, gamma, beta):
    raise NotImplementedError("write your pallas kernel here")


# speedup vs baseline: 1.4836x; 1.4836x over previous
import functools

import jax
import jax.numpy as jnp
from jax import lax
from jax.experimental import pallas as pl
from jax.experimental.pallas import tpu as pltpu

_B, _N = 4, 150000
_VX = _VY = 0.004
_X0 = _Y0 = 0.0
_NX = _NY = 250
_P = _NY * _NX
_C_OUT = 64


def _dense_body(acc_ref, wgA_ref, cc_ref, out_ref):
    s = acc_ref[0, 0]
    for k in range(1, acc_ref.shape[0]):
        s = s + acc_ref[k, 0]
    cnt = s[4:5, :]
    occ = (cnt > 0.0).astype(jnp.float32)
    inv = 1.0 / jnp.maximum(cnt, 1.0)
    mean4 = s[0:4, :] * inv
    cxy = cc_ref[...] * occ
    feats = jnp.concatenate([mean4, occ, cxy], axis=0)
    h = lax.dot_general(wgA_ref[...], feats, (((1,), (0,)), ((), ())),
                        preferred_element_type=jnp.float32)
    out_ref[0] = jnp.maximum(h, 0.0)


def _dense_stage(acc, W, b, gamma, beta):
    nS = acc.shape[0]
    wg = W[0:4, :] * gamma[None, :]
    wg = wg.at[0].add(W[7] * gamma)
    wg = wg.at[1].add(W[8] * gamma)
    b0 = b * gamma + beta
    wx = -W[7] * gamma
    wy = -W[8] * gamma
    wgA = jnp.concatenate(
        [wg, b0[None, :], wx[None, :], wy[None, :]], axis=0).T
    p = jnp.arange(_P, dtype=jnp.int32)
    cx = _X0 + ((p % _NX).astype(jnp.float32) + 0.5) * _VX
    cy = _Y0 + ((p // _NX).astype(jnp.float32) + 0.5) * _VY
    cc = jnp.stack([cx, cy], axis=0)

    out = pl.pallas_call(
        _dense_body,
        grid=(_B,),
        in_specs=[
            pl.BlockSpec((nS, 1, 5, _P), lambda bb: (0, bb, 0, 0)),
            pl.BlockSpec((_C_OUT, 7), lambda bb: (0, 0)),
            pl.BlockSpec((2, _P), lambda bb: (0, 0)),
        ],
        out_specs=pl.BlockSpec((1, _C_OUT, _P), lambda bb: (bb, 0, 0)),
        out_shape=jax.ShapeDtypeStruct((_B, _C_OUT, _P), jnp.float32),
    )(acc, wgA, cc)
    return out.reshape(_B, _C_OUT, _NY, _NX)


def _segment_acc_xla(points):
    x = points[..., 0]
    y = points[..., 1]
    ix = jnp.floor((x - _X0) / _VX).astype(jnp.int32)
    iy = jnp.floor((y - _Y0) / _VY).astype(jnp.int32)
    valid = (ix >= 0) & (ix < _NX) & (iy >= 0) & (iy < _NY)
    batch_ids = jnp.broadcast_to(
        jnp.arange(_B, dtype=jnp.int32)[:, None], (_B, _N))
    seg = batch_ids * _P + iy * _NX + ix
    total = _B * _P
    seg = jnp.where(valid, seg, total)
    m = valid.reshape(-1).astype(jnp.float32)
    vals = jnp.concatenate(
        [points.reshape(-1, 4) * m[:, None], m[:, None]], axis=1)
    sums = jax.ops.segment_sum(vals, seg.reshape(-1), num_segments=total + 1)[:total]
    return sums.reshape(_B, _P, 5).transpose(0, 2, 1)[None]


def kernel(points, W, b, gamma, beta):
    acc = _segment_acc_xla(points)
    return _dense_stage(acc, W, b, gamma, beta)


# speedup vs baseline: 2.9712x; 2.0028x over previous
import functools

import jax
import jax.numpy as jnp
from jax import lax
from jax.experimental import pallas as pl
from jax.experimental.pallas import tpu as pltpu
from jax.experimental.pallas import tpu_sc as plsc

_B, _N = 4, 150000
_VX = _VY = 0.004
_X0 = _Y0 = 0.0
_NX = _NY = 250
_P = _NY * _NX
_C_OUT = 64

_NC, _NS = 2, 16
_NW = _NC * _NS
_NPAD = 153600
_PPT = _B * _NPAD // _NW
_CK = 1920
_NCK = _PPT // _CK
_G = _CK // 128
_PLANE = 64000
_ROWS = 2 * _PLANE + 128
_ZR = _ROWS // _NS
_DUMP = 2 * _PLANE + 32
_TR = 1280
_NTCH = 2 * _PLANE // _TR


def _sc_body(pts_hbm, idx_hbm, zeros_hbm, out_hbm, vals, idx2, tbuf, tbufT, acc):
    c = lax.axis_index("c")
    s = lax.axis_index("s")
    wid = c * _NS + s
    iota = lax.iota(jnp.int32, 16)

    pltpu.sync_copy(zeros_hbm, acc.at[pl.ds(s * _ZR, _ZR), :])
    plsc.subcore_barrier()

    base_pt = wid * _PPT

    def chunk_body(ck, _):
        a = base_pt + ck * _CK
        pltpu.sync_copy(pts_hbm.at[pl.ds(a, _CK), :], vals)
        pltpu.sync_copy(idx_hbm.at[pl.ds(a // 128, _G), :], idx2)

        def stream_body(g, _):
            pltpu.sync_copy(vals.at[pl.ds(g * 128, 128), :],
                            acc.at[idx2.at[g]], add=True)
            return 0
        lax.fori_loop(0, _G, stream_body, 0)
        return 0
    lax.fori_loop(0, _NCK, chunk_body, 0)
    plsc.subcore_barrier()

    for t in range(-(-_NTCH // _NS)):
        k = s + t * _NS

        @pl.when(k < _NTCH)
        def _():
            row0 = k * _TR
            plane = k // (_PLANE // _TR)
            p0 = (k % (_PLANE // _TR)) * _TR
            pltpu.sync_copy(acc.at[pl.ds(row0, _TR), :], tbuf)
            for ch in range(5):
                def col_body(j, _):
                    v = plsc.load_gather(
                        tbuf, [j * 16 + iota, jnp.full((16,), ch, jnp.int32)])
                    tbufT[ch, pl.ds(j * 16, 16)] = v
                    return 0
                lax.fori_loop(0, _TR // 16, col_body, 0)
            pltpu.sync_copy(tbufT, out_hbm.at[c, plane, :, pl.ds(p0, _TR)])


def _sc_segment_acc(points):
    ones = jnp.ones((_B, _N, 1), jnp.float32)
    zer3 = jnp.zeros((_B, _N, 3), jnp.float32)
    pts8 = jnp.concatenate([points, ones, zer3], axis=2)
    pad = jnp.full((_B, _NPAD - _N, 8), -1.0, jnp.float32)
    pts8 = jnp.concatenate([pts8, pad], axis=1).reshape(_B * _NPAD, 8)
    x = pts8[:, 0]
    y = pts8[:, 1]
    ix = jnp.floor((x - _X0) / jnp.float32(_VX)).astype(jnp.int32)
    iy = jnp.floor((y - _Y0) / jnp.float32(_VY)).astype(jnp.int32)
    valid = (ix >= 0) & (ix < _NX) & (iy >= 0) & (iy < _NY)
    b_loc = (jnp.arange(_B * _NPAD, dtype=jnp.int32) // _NPAD) % 2
    seg = b_loc * _PLANE + iy * _NX + ix
    seg = jnp.where(valid, seg, _DUMP)
    idx = seg.reshape(_B * _NPAD // 128, 128)
    zeros = jnp.zeros((_ZR, 8), jnp.float32)

    mesh = plsc.VectorSubcoreMesh(core_axis_name="c", subcore_axis_name="s",
                                  num_cores=_NC, num_subcores=_NS)
    out = pl.kernel(
        _sc_body,
        out_type=jax.ShapeDtypeStruct((_NC, 2, 5, _PLANE), jnp.float32),
        mesh=mesh,
        compiler_params=pltpu.CompilerParams(
            needs_layout_passes=False, use_tc_tiling_on_sc=False),
        scratch_types=[
            pltpu.VMEM((_CK, 8), jnp.float32),
            pltpu.VMEM((_G, 128), jnp.int32),
            pltpu.VMEM((_TR, 8), jnp.float32),
            pltpu.VMEM((5, _TR), jnp.float32),
            pltpu.VMEM_SHARED((_ROWS, 8), jnp.float32),
        ],
    )(pts8, idx, zeros)
    return out.reshape(_B, 5, _PLANE)


def _dense_body(acc_ref, wgA_ref, cc_ref, out_ref):
    s = acc_ref[0][:, :_P]
    cnt = s[4:5, :]
    occ = (cnt > 0.0).astype(jnp.float32)
    inv = 1.0 / jnp.maximum(cnt, 1.0)
    mean4 = s[0:4, :] * inv
    cxy = cc_ref[...] * occ
    feats = jnp.concatenate([mean4, occ, cxy], axis=0)
    h = lax.dot_general(wgA_ref[...], feats, (((1,), (0,)), ((), ())),
                        preferred_element_type=jnp.float32)
    out_ref[0] = jnp.maximum(h, 0.0)


def _dense_stage(acc, W, b, gamma, beta):
    wg = W[0:4, :] * gamma[None, :]
    wg = wg.at[0].add(W[7] * gamma)
    wg = wg.at[1].add(W[8] * gamma)
    b0 = b * gamma + beta
    wx = -W[7] * gamma
    wy = -W[8] * gamma
    wgA = jnp.concatenate(
        [wg, b0[None, :], wx[None, :], wy[None, :]], axis=0).T
    p = jnp.arange(_P, dtype=jnp.int32)
    cx = _X0 + ((p % _NX).astype(jnp.float32) + 0.5) * _VX
    cy = _Y0 + ((p // _NX).astype(jnp.float32) + 0.5) * _VY
    cc = jnp.stack([cx, cy], axis=0)

    out = pl.pallas_call(
        _dense_body,
        grid=(_B,),
        in_specs=[
            pl.BlockSpec((1, 5, _PLANE), lambda bb: (bb, 0, 0)),
            pl.BlockSpec((_C_OUT, 7), lambda bb: (0, 0)),
            pl.BlockSpec((2, _P), lambda bb: (0, 0)),
        ],
        out_specs=pl.BlockSpec((1, _C_OUT, _P), lambda bb: (bb, 0, 0)),
        out_shape=jax.ShapeDtypeStruct((_B, _C_OUT, _P), jnp.float32),
    )(acc, wgA, cc)
    return out.reshape(_B, _C_OUT, _NY, _NX)


def kernel(points, W, b, gamma, beta):
    acc = _sc_segment_acc(points)
    return _dense_stage(acc, W, b, gamma, beta)


# speedup vs baseline: 3.4694x; 1.1676x over previous
import functools

import jax
import jax.numpy as jnp
from jax import lax
from jax.experimental import pallas as pl
from jax.experimental.pallas import tpu as pltpu
from jax.experimental.pallas import tpu_sc as plsc

_B, _N = 4, 150000
_VX = _VY = 0.004
_X0 = _Y0 = 0.0
_NX = _NY = 250
_C_OUT = 64

_NC, _NS = 2, 16
_NW = _NC * _NS
_NPAD = 153600
_PPT = _B * _NPAD // _NW
_CK = 1920
_NCK = _PPT // _CK
_G = _CK // 128
_XS = 256
_PLANE = 65536
_ROWS = 2 * _PLANE + 128
_ZR = _ROWS // _NS
_DUMP = 2 * _PLANE + 32
_TR = 1024
_NTCH = 2 * _PLANE // _TR


def _sc_body(pts_hbm, idx_hbm, out_hbm, pbuf, vals, idx2, tbuf, tbufT, acc):
    c = lax.axis_index("c")
    s = lax.axis_index("s")
    wid = c * _NS + s
    iota = lax.iota(jnp.int32, 16)

    def zb(j, _):
        rows = (j * 16 + iota) // 8
        cols = (j * 16 + iota) % 8
        plsc.store_scatter(tbuf, [rows, cols], jnp.zeros((16,), jnp.float32))
        return 0
    lax.fori_loop(0, _TR * 8 // 16, zb, 0)

    def fill_const(j, _):
        rows = j * 16 + iota
        plsc.store_scatter(vals, [rows, jnp.full((16,), 4, jnp.int32)],
                           jnp.full((16,), 1.0, jnp.float32))
        for cc in (5, 6, 7):
            plsc.store_scatter(vals, [rows, jnp.full((16,), cc, jnp.int32)],
                               jnp.zeros((16,), jnp.float32))
        return 0
    lax.fori_loop(0, _CK // 16, fill_const, 0)

    def zc(j, _):
        pltpu.sync_copy(tbuf, acc.at[pl.ds(s * _ZR + j * _TR, _TR), :])
        return 0
    lax.fori_loop(0, _ZR // _TR, zc, 0)
    _REM = _ZR % _TR
    if _REM:
        pltpu.sync_copy(tbuf.at[pl.ds(0, _REM), :],
                        acc.at[pl.ds(s * _ZR + (_ZR // _TR) * _TR, _REM), :])
    plsc.subcore_barrier()

    base_pt = wid * _PPT

    def chunk_body(ck, _):
        a = base_pt + ck * _CK
        pltpu.sync_copy(pts_hbm.at[pl.ds(a * 4, _CK * 4)], pbuf)
        pltpu.sync_copy(idx_hbm.at[pl.ds(a // 128, _G), :], idx2)

        def build_body(j, _):
            rows = j * 16 + iota
            r4 = rows * 4
            for cc in range(4):
                v = plsc.load_gather(pbuf, [r4 + cc])
                plsc.store_scatter(vals, [rows, jnp.full((16,), cc, jnp.int32)], v)
            return 0
        lax.fori_loop(0, _CK // 16, build_body, 0)

        def stream_body(g, _):
            pltpu.sync_copy(vals.at[pl.ds(g * 128, 128), :],
                            acc.at[idx2.at[g]], add=True)
            return 0
        lax.fori_loop(0, _G, stream_body, 0)
        return 0
    lax.fori_loop(0, _NCK, chunk_body, 0)
    plsc.subcore_barrier()

    for t in range(_NTCH // _NS):
        k = s * (_NTCH // _NS) + t
        row0 = k * _TR
        plane = k // (_PLANE // _TR)
        r0 = (k % (_PLANE // _TR)) * 8
        pltpu.sync_copy(acc.at[pl.ds(row0, _TR), :], tbuf)
        for ch in range(5):
            for rr in range(8):
                def col_body(j, _):
                    v = plsc.load_gather(
                        tbuf, [rr * 128 + j * 16 + iota,
                               jnp.full((16,), ch, jnp.int32)])
                    tbufT[ch, rr, pl.ds(j * 16, 16)] = v
                    return 0
                lax.fori_loop(0, 8, col_body, 0)
        pltpu.sync_copy(tbufT, out_hbm.at[c, plane, :, pl.ds(r0, 8), :])


def _sc_segment_acc(points):
    pf = points.reshape(_B, _N * 4)
    pf = jnp.concatenate(
        [pf, jnp.full((_B, (_NPAD - _N) * 4), -1.0, jnp.float32)], axis=1)
    pf = pf.reshape(_B * _NPAD * 4)
    x = points[:, :, 0]
    y = points[:, :, 1]
    ix = jnp.floor((x - _X0) / jnp.float32(_VX)).astype(jnp.int32)
    iy = jnp.floor((y - _Y0) / jnp.float32(_VY)).astype(jnp.int32)
    valid = (ix >= 0) & (ix < _NX) & (iy >= 0) & (iy < _NY)
    b_loc = (jnp.arange(_B, dtype=jnp.int32) % 2)[:, None]
    seg = b_loc * _PLANE + iy * _XS + ix
    seg = jnp.where(valid, seg, _DUMP)
    seg = jnp.concatenate(
        [seg, jnp.full((_B, _NPAD - _N), _DUMP, jnp.int32)], axis=1)
    idx = seg.reshape(_B * _NPAD // 128, 128)

    mesh = plsc.VectorSubcoreMesh(core_axis_name="c", subcore_axis_name="s",
                                  num_cores=_NC, num_subcores=_NS)
    out = pl.kernel(
        _sc_body,
        out_type=jax.ShapeDtypeStruct((_NC, 2, 5, _PLANE // 128, 128),
                                      jnp.float32),
        mesh=mesh,
        compiler_params=pltpu.CompilerParams(
            needs_layout_passes=False, use_tc_tiling_on_sc=False),
        scratch_types=[
            pltpu.VMEM((_CK * 4,), jnp.float32),
            pltpu.VMEM((_CK, 8), jnp.float32),
            pltpu.VMEM((_G, 128), jnp.int32),
            pltpu.VMEM((_TR, 8), jnp.float32),
            pltpu.VMEM((5, 8, 128), jnp.float32),
            pltpu.VMEM_SHARED((_ROWS, 8), jnp.float32),
        ],
    )(pf, idx)
    return out.reshape(_B, 5, _PLANE // 128, 128)


def _dense_body(acc_ref, wgA_ref, cc_ref, out_ref):
    s = acc_ref[0].reshape(5, _PLANE)
    cnt = s[4:5, :]
    occ = (cnt > 0.0).astype(jnp.float32)
    inv = 1.0 / jnp.maximum(cnt, 1.0)
    mean4 = s[0:4, :] * inv
    cxy = cc_ref[...].reshape(2, _PLANE) * occ
    feats = jnp.concatenate([mean4, occ, cxy], axis=0)
    h = lax.dot_general(wgA_ref[...], feats, (((1,), (0,)), ((), ())),
                        preferred_element_type=jnp.float32)
    h = jnp.maximum(h, 0.0).reshape(_C_OUT, _PLANE // _XS, _XS)
    out_ref[0] = h[:, :_NY, :_NX]


def _dense_stage(acc, W, b, gamma, beta):
    wg = W[0:4, :] * gamma[None, :]
    wg = wg.at[0].add(W[7] * gamma)
    wg = wg.at[1].add(W[8] * gamma)
    b0 = b * gamma + beta
    wx = -W[7] * gamma
    wy = -W[8] * gamma
    wgA = jnp.concatenate(
        [wg, b0[None, :], wx[None, :], wy[None, :]], axis=0).T
    p = jnp.arange(_PLANE, dtype=jnp.int32)
    cx = _X0 + ((p % _XS).astype(jnp.float32) + 0.5) * _VX
    cy = _Y0 + ((p // _XS).astype(jnp.float32) + 0.5) * _VY
    cc = jnp.stack([cx, cy], axis=0).reshape(2, _PLANE // 128, 128)

    return pl.pallas_call(
        _dense_body,
        grid=(_B,),
        in_specs=[
            pl.BlockSpec((1, 5, _PLANE // 128, 128), lambda bb: (bb, 0, 0, 0)),
            pl.BlockSpec((_C_OUT, 7), lambda bb: (0, 0)),
            pl.BlockSpec((2, _PLANE // 128, 128), lambda bb: (0, 0, 0)),
        ],
        out_specs=pl.BlockSpec((1, _C_OUT, _NY, _NX), lambda bb: (bb, 0, 0, 0)),
        out_shape=jax.ShapeDtypeStruct((_B, _C_OUT, _NY, _NX), jnp.float32),
    )(acc, wgA, cc)


def kernel(points, W, b, gamma, beta):
    acc = _sc_segment_acc(points)
    return _dense_stage(acc, W, b, gamma, beta)


# speedup vs baseline: 14.7173x; 4.2421x over previous
import functools

import jax
import jax.numpy as jnp
from jax import lax
from jax.experimental import pallas as pl
from jax.experimental.pallas import tpu as pltpu
from jax.experimental.pallas import tpu_sc as plsc

_B, _N = 4, 150000
_VX = _VY = 0.004
_X0 = _Y0 = 0.0
_NX = _NY = 250
_C_OUT = 64

_NC, _NS = 2, 16
_NW = _NC * _NS
_NPAD = 153600
_PPT = _B * _NPAD // _NW
_CK = 1920
_NCK = _PPT // _CK
_G = _CK // 128
_XS = 256
_PLANE = 65536
_ROWS = 2 * _PLANE + 128
_ZR = _ROWS // _NS
_DUMP = 2 * _PLANE + 32
_TR = 1024
_NTCH = 2 * _PLANE // _TR


def _sc_body(pts_hbm, idx_hbm, out_hbm, pbuf, vals, idx2, tbuf, tbufT, acc):
    c = lax.axis_index("c")
    s = lax.axis_index("s")
    wid = c * _NS + s
    bb = wid // 8
    o0 = (wid % 8) * _PPT
    iota = lax.iota(jnp.int32, 16)

    def zb(j, _):
        rows = (j * 16 + iota) // 8
        cols = (j * 16 + iota) % 8
        plsc.store_scatter(tbuf, [rows, cols], jnp.zeros((16,), jnp.float32))
        return 0
    lax.fori_loop(0, _TR * 8 // 16, zb, 0)

    def fill_const(j, _):
        rows = j * 16 + iota
        plsc.store_scatter(vals, [rows, jnp.full((16,), 4, jnp.int32)],
                           jnp.full((16,), 1.0, jnp.float32))
        for cc in (5, 6, 7):
            plsc.store_scatter(vals, [rows, jnp.full((16,), cc, jnp.int32)],
                               jnp.zeros((16,), jnp.float32))
        return 0
    lax.fori_loop(0, _CK // 16, fill_const, 0)

    def zc(j, _):
        pltpu.sync_copy(tbuf, acc.at[pl.ds(s * _ZR + j * _TR, _TR), :])
        return 0
    lax.fori_loop(0, _ZR // _TR, zc, 0)
    _REM = _ZR % _TR
    if _REM:
        pltpu.sync_copy(tbuf.at[pl.ds(0, _REM), :],
                        acc.at[pl.ds(s * _ZR + (_ZR // _TR) * _TR, _REM), :])
    plsc.subcore_barrier()

    base_pt = wid * _PPT

    def chunk_body(ck, _):
        a = base_pt + ck * _CK
        for cc in range(4):
            pltpu.sync_copy(
                pts_hbm.at[bb * 4 + cc, pl.ds(o0 + ck * _CK, _CK)],
                pbuf.at[pl.ds(cc * _CK, _CK)])
        pltpu.sync_copy(idx_hbm.at[pl.ds(a // 128, _G), :], idx2)

        def build_body(j, _):
            rows = j * 16 + iota
            for cc in range(4):
                v = pbuf[pl.ds(cc * _CK + j * 16, 16)]
                plsc.store_scatter(vals, [rows, jnp.full((16,), cc, jnp.int32)], v)
            return 0
        lax.fori_loop(0, _CK // 16, build_body, 0)

        def stream_body(g, _):
            pltpu.sync_copy(vals.at[pl.ds(g * 128, 128), :],
                            acc.at[idx2.at[g]], add=True)
            return 0
        lax.fori_loop(0, _G, stream_body, 0)
        return 0
    lax.fori_loop(0, _NCK, chunk_body, 0)
    plsc.subcore_barrier()

    for t in range(_NTCH // _NS):
        k = s * (_NTCH // _NS) + t
        row0 = k * _TR
        plane = k // (_PLANE // _TR)
        r0 = (k % (_PLANE // _TR)) * 8
        pltpu.sync_copy(acc.at[pl.ds(row0, _TR), :], tbuf)
        for ch in range(5):
            for rr in range(8):
                def col_body(j, _):
                    v = plsc.load_gather(
                        tbuf, [rr * 128 + j * 16 + iota,
                               jnp.full((16,), ch, jnp.int32)])
                    tbufT[ch, rr, pl.ds(j * 16, 16)] = v
                    return 0
                lax.fori_loop(0, 8, col_body, 0)
        pltpu.sync_copy(tbufT, out_hbm.at[c, plane, :, pl.ds(r0, 8), :])


def _sc_segment_acc(points):
    pf = points.transpose(0, 2, 1).reshape(_B * 4, _N)
    pf = jnp.concatenate(
        [pf, jnp.full((_B * 4, _NPAD - _N), -1.0, jnp.float32)], axis=1)
    x = points[:, :, 0]
    y = points[:, :, 1]
    ix = jnp.floor((x - _X0) / jnp.float32(_VX)).astype(jnp.int32)
    iy = jnp.floor((y - _Y0) / jnp.float32(_VY)).astype(jnp.int32)
    valid = (ix >= 0) & (ix < _NX) & (iy >= 0) & (iy < _NY)
    b_loc = (jnp.arange(_B, dtype=jnp.int32) % 2)[:, None]
    seg = b_loc * _PLANE + iy * _XS + ix
    seg = jnp.where(valid, seg, _DUMP)
    seg = jnp.concatenate(
        [seg, jnp.full((_B, _NPAD - _N), _DUMP, jnp.int32)], axis=1)
    idx = seg.reshape(_B * _NPAD // 128, 128)

    mesh = plsc.VectorSubcoreMesh(core_axis_name="c", subcore_axis_name="s",
                                  num_cores=_NC, num_subcores=_NS)
    out = pl.kernel(
        _sc_body,
        out_type=jax.ShapeDtypeStruct((_NC, 2, 5, _PLANE // 128, 128),
                                      jnp.float32),
        mesh=mesh,
        compiler_params=pltpu.CompilerParams(
            needs_layout_passes=False, use_tc_tiling_on_sc=False),
        scratch_types=[
            pltpu.VMEM((_CK * 4,), jnp.float32),
            pltpu.VMEM((_CK, 8), jnp.float32),
            pltpu.VMEM((_G, 128), jnp.int32),
            pltpu.VMEM((_TR, 8), jnp.float32),
            pltpu.VMEM((5, 8, 128), jnp.float32),
            pltpu.VMEM_SHARED((_ROWS, 8), jnp.float32),
        ],
    )(pf, idx)
    return out.reshape(_B, 5, _PLANE // 128, 128)


def _dense_body(acc_ref, wgA_ref, cc_ref, out_ref):
    s = acc_ref[0].reshape(5, _PLANE)
    cnt = s[4:5, :]
    occ = (cnt > 0.0).astype(jnp.float32)
    inv = 1.0 / jnp.maximum(cnt, 1.0)
    mean4 = s[0:4, :] * inv
    cxy = cc_ref[...].reshape(2, _PLANE) * occ
    feats = jnp.concatenate([mean4, occ, cxy], axis=0)
    h = lax.dot_general(wgA_ref[...], feats, (((1,), (0,)), ((), ())),
                        preferred_element_type=jnp.float32)
    h = jnp.maximum(h, 0.0).reshape(_C_OUT, _PLANE // _XS, _XS)
    out_ref[0] = h[:, :_NY, :_NX]


def _dense_stage(acc, W, b, gamma, beta):
    wg = W[0:4, :] * gamma[None, :]
    wg = wg.at[0].add(W[7] * gamma)
    wg = wg.at[1].add(W[8] * gamma)
    b0 = b * gamma + beta
    wx = -W[7] * gamma
    wy = -W[8] * gamma
    wgA = jnp.concatenate(
        [wg, b0[None, :], wx[None, :], wy[None, :]], axis=0).T
    p = jnp.arange(_PLANE, dtype=jnp.int32)
    cx = _X0 + ((p % _XS).astype(jnp.float32) + 0.5) * _VX
    cy = _Y0 + ((p // _XS).astype(jnp.float32) + 0.5) * _VY
    cc = jnp.stack([cx, cy], axis=0).reshape(2, _PLANE // 128, 128)

    return pl.pallas_call(
        _dense_body,
        grid=(_B,),
        in_specs=[
            pl.BlockSpec((1, 5, _PLANE // 128, 128), lambda bb: (bb, 0, 0, 0)),
            pl.BlockSpec((_C_OUT, 7), lambda bb: (0, 0)),
            pl.BlockSpec((2, _PLANE // 128, 128), lambda bb: (0, 0, 0)),
        ],
        out_specs=pl.BlockSpec((1, _C_OUT, _NY, _NX), lambda bb: (bb, 0, 0, 0)),
        out_shape=jax.ShapeDtypeStruct((_B, _C_OUT, _NY, _NX), jnp.float32),
    )(acc, wgA, cc)


def kernel(points, W, b, gamma, beta):
    acc = _sc_segment_acc(points)
    return _dense_stage(acc, W, b, gamma, beta)


# speedup vs baseline: 15.1941x; 1.0324x over previous
import functools

import jax
import jax.numpy as jnp
from jax import lax
from jax.experimental import pallas as pl
from jax.experimental.pallas import tpu as pltpu
from jax.experimental.pallas import tpu_sc as plsc

_B, _N = 4, 150000
_VX = _VY = 0.004
_X0 = _Y0 = 0.0
_NX = _NY = 250
_C_OUT = 64

_NC, _NS = 2, 16
_NW = _NC * _NS
_NPAD = 153600
_PPT = _B * _NPAD // _NW
_CK = 1920
_NCK = _PPT // _CK
_G = _CK // 128
_XS = 256
_PLANE = 65536
_ROWS = 2 * _PLANE + 128
_ZR = _ROWS // _NS
_DUMP = 2 * _PLANE + 32
_TR = 1024
_NTCH = 2 * _PLANE // _TR


def _sc_body(pts_hbm, idx_hbm, out_hbm, pbuf, vals, idx2, tbuf, tbufT, acc):
    c = lax.axis_index("c")
    s = lax.axis_index("s")
    wid = c * _NS + s
    bb = wid // 8
    o0 = (wid % 8) * _PPT
    iota = lax.iota(jnp.int32, 16)

    def zb(j, _):
        rows = (j * 16 + iota) // 8
        cols = (j * 16 + iota) % 8
        plsc.store_scatter(tbuf, [rows, cols], jnp.zeros((16,), jnp.float32))
        return 0
    lax.fori_loop(0, _TR * 8 // 16, zb, 0)

    def fill_const(j, _):
        rows = j * 16 + iota
        plsc.store_scatter(vals, [rows, jnp.full((16,), 4, jnp.int32)],
                           jnp.full((16,), 1.0, jnp.float32))
        for cc in (5, 6, 7):
            plsc.store_scatter(vals, [rows, jnp.full((16,), cc, jnp.int32)],
                               jnp.zeros((16,), jnp.float32))
        return 0
    lax.fori_loop(0, _CK // 16, fill_const, 0)

    def zc(j, _):
        pltpu.sync_copy(tbuf, acc.at[pl.ds(s * _ZR + j * _TR, _TR), :])
        return 0
    lax.fori_loop(0, _ZR // _TR, zc, 0)
    _REM = _ZR % _TR
    if _REM:
        pltpu.sync_copy(tbuf.at[pl.ds(0, _REM), :],
                        acc.at[pl.ds(s * _ZR + (_ZR // _TR) * _TR, _REM), :])
    plsc.subcore_barrier()

    base_pt = wid * _PPT

    def chunk_body(ck, _):
        a = base_pt + ck * _CK
        for cc in range(4):
            pltpu.sync_copy(
                pts_hbm.at[bb * 4 + cc, pl.ds(o0 + ck * _CK, _CK)],
                pbuf.at[pl.ds(cc * _CK, _CK)])
        pltpu.sync_copy(idx_hbm.at[pl.ds(a // 128, _G), :], idx2)

        def build_body(j, _):
            rows = j * 16 + iota
            for cc in range(4):
                v = pbuf[pl.ds(cc * _CK + j * 16, 16)]
                plsc.store_scatter(vals, [rows, jnp.full((16,), cc, jnp.int32)], v)
            return 0
        lax.fori_loop(0, _CK // 16, build_body, 0)

        def stream_body(g, _):
            pltpu.sync_copy(vals.at[pl.ds(g * 128, 128), :],
                            acc.at[idx2.at[g]], add=True)
            return 0
        lax.fori_loop(0, _G, stream_body, 0)
        return 0
    lax.fori_loop(0, _NCK, chunk_body, 0)
    plsc.subcore_barrier()

    for t in range(_NTCH // _NS):
        k = s * (_NTCH // _NS) + t
        row0 = k * _TR
        plane = k // (_PLANE // _TR)
        r0 = (k % (_PLANE // _TR)) * 8
        pltpu.sync_copy(acc.at[pl.ds(row0, _TR), :], tbuf)
        for ch in range(5):
            for rr in range(8):
                def col_body(j, _):
                    v = plsc.load_gather(
                        tbuf, [rr * 128 + j * 16 + iota,
                               jnp.full((16,), ch, jnp.int32)])
                    tbufT[ch, rr, pl.ds(j * 16, 16)] = v
                    return 0
                lax.fori_loop(0, 8, col_body, 0)
        pltpu.sync_copy(tbufT, out_hbm.at[c, plane, :, pl.ds(r0, 8), :])


def _sc_segment_acc(points):
    pf = points.transpose(0, 2, 1).reshape(_B * 4, _N)
    pf = jnp.concatenate(
        [pf, jnp.full((_B * 4, _NPAD - _N), -1.0, jnp.float32)], axis=1)
    x = points[:, :, 0]
    y = points[:, :, 1]
    ix = jnp.floor((x - _X0) / jnp.float32(_VX)).astype(jnp.int32)
    iy = jnp.floor((y - _Y0) / jnp.float32(_VY)).astype(jnp.int32)
    valid = (ix >= 0) & (ix < _NX) & (iy >= 0) & (iy < _NY)
    b_loc = (jnp.arange(_B, dtype=jnp.int32) % 2)[:, None]
    seg = b_loc * _PLANE + iy * _XS + ix
    seg = jnp.where(valid, seg, _DUMP)
    seg = jnp.concatenate(
        [seg, jnp.full((_B, _NPAD - _N), _DUMP, jnp.int32)], axis=1)
    idx = seg.reshape(_B * _NPAD // 128, 128)

    mesh = plsc.VectorSubcoreMesh(core_axis_name="c", subcore_axis_name="s",
                                  num_cores=_NC, num_subcores=_NS)
    out = pl.kernel(
        _sc_body,
        out_type=jax.ShapeDtypeStruct((_NC, 2, 5, _PLANE // 128, 128),
                                      jnp.float32),
        mesh=mesh,
        compiler_params=pltpu.CompilerParams(
            needs_layout_passes=False, use_tc_tiling_on_sc=False),
        scratch_types=[
            pltpu.VMEM((_CK * 4,), jnp.float32),
            pltpu.VMEM((_CK, 8), jnp.float32),
            pltpu.VMEM((_G, 128), jnp.int32),
            pltpu.VMEM((_TR, 8), jnp.float32),
            pltpu.VMEM((5, 8, 128), jnp.float32),
            pltpu.VMEM_SHARED((_ROWS, 8), jnp.float32),
        ],
    )(pf, idx)
    return out.reshape(_B, 5, _PLANE // 128, 128)


_YT = 10


def _dense_body(acc_ref, wgA_ref, cc_ref, out_ref):
    s = acc_ref[0]
    cnt = s[4:5, :]
    occ = (cnt > 0.0).astype(jnp.float32)
    inv = 1.0 / jnp.maximum(cnt, 1.0)
    mean4 = s[0:4, :] * inv
    cxy = cc_ref[...] * occ
    feats = jnp.concatenate([mean4, occ, cxy], axis=0)
    f3 = feats.reshape(7, _YT, _XS)
    for y in range(_YT):
        h = lax.dot_general(wgA_ref[...], f3[:, y, :], (((1,), (0,)), ((), ())),
                            preferred_element_type=jnp.float32)
        out_ref[0, y] = jnp.maximum(h, 0.0)[:, :_NX]


def _dense_stage(acc, W, b, gamma, beta):
    wg = W[0:4, :] * gamma[None, :]
    wg = wg.at[0].add(W[7] * gamma)
    wg = wg.at[1].add(W[8] * gamma)
    b0 = b * gamma + beta
    wx = -W[7] * gamma
    wy = -W[8] * gamma
    wgA = jnp.concatenate(
        [wg, b0[None, :], wx[None, :], wy[None, :]], axis=0).T
    p = jnp.arange(_PLANE, dtype=jnp.int32)
    cx = _X0 + ((p % _XS).astype(jnp.float32) + 0.5) * _VX
    cy = _Y0 + ((p // _XS).astype(jnp.float32) + 0.5) * _VY
    cc = jnp.stack([cx, cy], axis=0)
    acc3 = acc.reshape(_B, 5, _PLANE)

    lb = _YT * _XS
    out = pl.pallas_call(
        _dense_body,
        grid=(_B, _NY // _YT),
        in_specs=[
            pl.BlockSpec((1, 5, lb), lambda bb, tt: (bb, 0, tt)),
            pl.BlockSpec((_C_OUT, 7), lambda bb, tt: (0, 0)),
            pl.BlockSpec((2, lb), lambda bb, tt: (0, tt)),
        ],
        out_specs=pl.BlockSpec((1, _YT, _C_OUT, _NX),
                               lambda bb, tt: (bb, tt, 0, 0)),
        out_shape=jax.ShapeDtypeStruct((_B, _NY, _C_OUT, _NX), jnp.float32),
    )(acc3, wgA, cc)
    return out.transpose(0, 2, 1, 3)


def kernel(points, W, b, gamma, beta):
    acc = _sc_segment_acc(points)
    return _dense_stage(acc, W, b, gamma, beta)


# speedup vs baseline: 17.9091x; 1.1787x over previous
import functools

import jax
import jax.numpy as jnp
from jax import lax
from jax.experimental import pallas as pl
from jax.experimental.pallas import tpu as pltpu
from jax.experimental.pallas import tpu_sc as plsc

_B, _N = 4, 150000
_VX = _VY = 0.004
_X0 = _Y0 = 0.0
_NX = _NY = 250
_C_OUT = 64

_NC, _NS = 2, 16
_NW = _NC * _NS
_NPAD = 153600
_PPT = _B * _NPAD // _NW
_CK = 1280
_NCK = _PPT // _CK
_G = _CK // 128
_XS = 256
_PLANE = 65536
_ROWS = 2 * _PLANE + 128
_ZR = _ROWS // _NS
_DUMP = 2 * _PLANE + 32
_TR = 512
_NTCH = 2 * _PLANE // _TR
_TPT = _NTCH // _NS


def _sc_body(pts_hbm, idx_hbm, out_hbm, pbuf, vals, idx2, tbuf, tbufT, acc,
             sem_in, sem_st, sem_z, sem_out):
    c = lax.axis_index("c")
    s = lax.axis_index("s")
    wid = c * _NS + s
    bb = wid // 8
    o0 = (wid % 8) * _PPT
    iota = lax.iota(jnp.int32, 16)

    def zb(j, _):
        f = j * 16 + iota
        plsc.store_scatter(tbuf, [f // (8 * _TR), (f // 8) % _TR, f % 8],
                           jnp.zeros((16,), jnp.float32))
        return 0
    lax.fori_loop(0, 2 * _TR * 8 // 16, zb, 0)

    def fill_const(j, _):
        f = j * 16 + iota
        for cc in (4, 5, 6, 7):
            plsc.store_scatter(
                vals, [f // _CK, f % _CK, jnp.full((16,), cc, jnp.int32)],
                jnp.full((16,), 1.0 if cc == 4 else 0.0, jnp.float32))
        return 0
    lax.fori_loop(0, 2 * _CK // 16, fill_const, 0)

    zcps = []
    for j in range(_ZR // _TR):
        zcps.append(pltpu.async_copy(
            tbuf.at[0], acc.at[pl.ds(s * _ZR + j * _TR, _TR), :], sem_z))
    _REM = _ZR % _TR
    if _REM:
        zcps.append(pltpu.async_copy(
            tbuf.at[0, pl.ds(0, _REM), :],
            acc.at[pl.ds(s * _ZR + (_ZR // _TR) * _TR, _REM), :], sem_z))
    for d in zcps:
        d.wait()
    plsc.subcore_barrier()

    def issue_in(ck):
        u = ck & 1
        ds = []
        for cc in range(4):
            ds.append(pltpu.async_copy(
                pts_hbm.at[bb * 4 + cc, pl.ds(o0 + ck * _CK, _CK)],
                pbuf.at[u, pl.ds(cc * _CK, _CK)], sem_in))
        a = (wid * _PPT + ck * _CK) // 128
        ds.append(pltpu.async_copy(idx_hbm.at[pl.ds(a, _G), :],
                                   idx2.at[u], sem_in))
        return ds

    indmas = {0: issue_in(0)}
    stdmas = {}
    for ck in range(_NCK):
        u = ck & 1
        for d in indmas.pop(ck):
            d.wait()
        if ck - 1 in stdmas:
            for d in stdmas.pop(ck - 1):
                d.wait()
        if ck + 1 < _NCK:
            indmas[ck + 1] = issue_in(ck + 1)

        def build_body(j, _, u=u):
            rows = j * 16 + iota
            for cc in range(4):
                v = pbuf[u, pl.ds(cc * _CK + j * 16, 16)]
                plsc.store_scatter(
                    vals, [jnp.full((16,), u, jnp.int32), rows,
                           jnp.full((16,), cc, jnp.int32)], v)
            return 0
        lax.fori_loop(0, _CK // 16, build_body, 0)

        sts = []
        for g in range(_G):
            sts.append(pltpu.async_copy(
                vals.at[u, pl.ds(g * 128, 128), :],
                acc.at[idx2.at[u, g]], sem_st, add=True))
        stdmas[ck] = sts
    for ck in sorted(stdmas):
        for d in stdmas[ck]:
            d.wait()
    plsc.subcore_barrier()

    def issue_tin(t):
        return pltpu.async_copy(
            acc.at[pl.ds((s * _TPT + t) * _TR, _TR), :], tbuf.at[t & 1],
            sem_in)

    tin = {0: issue_tin(0)}
    touts = {}
    for t in range(_TPT):
        u = t & 1
        k = s * _TPT + t
        plane = k // (_PLANE // _TR)
        r0 = (k % (_PLANE // _TR)) * (_TR // 128)
        tin.pop(t).wait()
        if t + 1 < _TPT:
            tin[t + 1] = issue_tin(t + 1)
        if t - 2 in touts:
            touts.pop(t - 2).wait()
        for ch in range(5):
            for rr in range(_TR // 128):
                def col_body(j, _, u=u, ch=ch, rr=rr):
                    v = plsc.load_gather(
                        tbuf, [jnp.full((16,), u, jnp.int32),
                               rr * 128 + j * 16 + iota,
                               jnp.full((16,), ch, jnp.int32)])
                    tbufT[u, ch, rr, pl.ds(j * 16, 16)] = v
                    return 0
                lax.fori_loop(0, 8, col_body, 0)
        touts[t] = pltpu.async_copy(
            tbufT.at[u],
            out_hbm.at[c, plane, :, pl.ds(r0, _TR // 128), :], sem_out)
    for t in sorted(touts):
        touts[t].wait()


def _sc_segment_acc(points):
    pf = points.transpose(0, 2, 1).reshape(_B * 4, _N)
    pf = jnp.concatenate(
        [pf, jnp.full((_B * 4, _NPAD - _N), -1.0, jnp.float32)], axis=1)
    x = points[:, :, 0]
    y = points[:, :, 1]
    ix = jnp.floor((x - _X0) / jnp.float32(_VX)).astype(jnp.int32)
    iy = jnp.floor((y - _Y0) / jnp.float32(_VY)).astype(jnp.int32)
    valid = (ix >= 0) & (ix < _NX) & (iy >= 0) & (iy < _NY)
    b_loc = (jnp.arange(_B, dtype=jnp.int32) % 2)[:, None]
    seg = b_loc * _PLANE + iy * _XS + ix
    seg = jnp.where(valid, seg, _DUMP)
    seg = jnp.concatenate(
        [seg, jnp.full((_B, _NPAD - _N), _DUMP, jnp.int32)], axis=1)
    idx = seg.reshape(_B * _NPAD // 128, 128)

    mesh = plsc.VectorSubcoreMesh(core_axis_name="c", subcore_axis_name="s",
                                  num_cores=_NC, num_subcores=_NS)
    out = pl.kernel(
        _sc_body,
        out_type=jax.ShapeDtypeStruct((_NC, 2, 5, _PLANE // 128, 128),
                                      jnp.float32),
        mesh=mesh,
        compiler_params=pltpu.CompilerParams(
            needs_layout_passes=False, use_tc_tiling_on_sc=False),
        scratch_types=[
            pltpu.VMEM((2, _CK * 4), jnp.float32),
            pltpu.VMEM((2, _CK, 8), jnp.float32),
            pltpu.VMEM((2, _G, 128), jnp.int32),
            pltpu.VMEM((2, _TR, 8), jnp.float32),
            pltpu.VMEM((2, 5, _TR // 128, 128), jnp.float32),
            pltpu.VMEM_SHARED((_ROWS, 8), jnp.float32),
            pltpu.SemaphoreType.DMA,
            pltpu.SemaphoreType.DMA,
            pltpu.SemaphoreType.DMA,
            pltpu.SemaphoreType.DMA,
        ],
    )(pf, idx)
    return out.reshape(_B, 5, _PLANE // 128, 128)


_YT = 10


def _dense_body(acc_ref, wgA_ref, cc_ref, out_ref):
    s = acc_ref[0]
    cnt = s[4:5, :]
    occ = (cnt > 0.0).astype(jnp.float32)
    inv = 1.0 / jnp.maximum(cnt, 1.0)
    mean4 = s[0:4, :] * inv
    cxy = cc_ref[...] * occ
    feats = jnp.concatenate([mean4, occ, cxy], axis=0)
    f3 = feats.reshape(7, _YT, _XS)
    for y in range(_YT):
        h = lax.dot_general(wgA_ref[...], f3[:, y, :], (((1,), (0,)), ((), ())),
                            preferred_element_type=jnp.float32)
        out_ref[0, y] = jnp.maximum(h, 0.0)[:, :_NX]


def _dense_stage(acc, W, b, gamma, beta):
    wg = W[0:4, :] * gamma[None, :]
    wg = wg.at[0].add(W[7] * gamma)
    wg = wg.at[1].add(W[8] * gamma)
    b0 = b * gamma + beta
    wx = -W[7] * gamma
    wy = -W[8] * gamma
    wgA = jnp.concatenate(
        [wg, b0[None, :], wx[None, :], wy[None, :]], axis=0).T
    p = jnp.arange(_PLANE, dtype=jnp.int32)
    cx = _X0 + ((p % _XS).astype(jnp.float32) + 0.5) * _VX
    cy = _Y0 + ((p // _XS).astype(jnp.float32) + 0.5) * _VY
    cc = jnp.stack([cx, cy], axis=0)
    acc3 = acc.reshape(_B, 5, _PLANE)

    lb = _YT * _XS
    out = pl.pallas_call(
        _dense_body,
        grid=(_B, _NY // _YT),
        in_specs=[
            pl.BlockSpec((1, 5, lb), lambda bb, tt: (bb, 0, tt)),
            pl.BlockSpec((_C_OUT, 7), lambda bb, tt: (0, 0)),
            pl.BlockSpec((2, lb), lambda bb, tt: (0, tt)),
        ],
        out_specs=pl.BlockSpec((1, _YT, _C_OUT, _NX),
                               lambda bb, tt: (bb, tt, 0, 0)),
        out_shape=jax.ShapeDtypeStruct((_B, _NY, _C_OUT, _NX), jnp.float32),
    )(acc3, wgA, cc)
    return out.transpose(0, 2, 1, 3)


def kernel(points, W, b, gamma, beta):
    acc = _sc_segment_acc(points)
    return _dense_stage(acc, W, b, gamma, beta)
